# baseline (speedup 1.0000x reference)
import functools

import jax
import jax.numpy as jnp
from jax import lax
from jax.experimental import pallas as pl
from jax.experimental.pallas import tpu as pltpu
from jax.experimental.pallas import tpu_sc as plsc

NUM_HEADS = 16
NEG = float(jnp.finfo(jnp.float32).min)


def _mask_body(dec_mask_ref, enc_mask_ref, ehs_ref,
               dec_out_ref, enc_out_ref, ehs_out_ref):
    i = pl.program_id(0)
    _, _, R, S = dec_out_ref.shape
    row = i * R + lax.broadcasted_iota(jnp.int32, (1, 1, R, S), 2)
    col = lax.broadcasted_iota(jnp.int32, (1, 1, R, S), 3)
    m = dec_mask_ref[0, :].astype(jnp.float32)[None, None, None, :]
    b = (1.0 - m) * NEG
    dec_out_ref[...] = jnp.where(col <= row, b, NEG)
    e = enc_mask_ref[0, :].astype(jnp.float32)[None, None, None, :]
    enc_out_ref[...] = (1.0 - e) * NEG
    ehs_out_ref[...] = ehs_ref[...]


def _make_masks(dec_mask, enc_mask, ehs_flat):
    _, s_dec = dec_mask.shape
    _, s_enc = enc_mask.shape
    n_ehs, d_model = ehs_flat.shape
    rows_per_step = 256
    grid = s_dec // rows_per_step
    er = n_ehs // grid
    return pl.pallas_call(
        _mask_body,
        grid=(grid,),
        in_specs=[
            pl.BlockSpec((1, s_dec), lambda i: (0, 0)),
            pl.BlockSpec((1, s_enc), lambda i: (0, 0)),
            pl.BlockSpec((er, d_model), lambda i: (i, 0)),
        ],
        out_specs=[
            pl.BlockSpec((1, 1, rows_per_step, s_dec), lambda i: (0, 0, i, 0)),
            pl.BlockSpec((1, 1, 1, s_enc), lambda i: (0, 0, 0, 0)),
            pl.BlockSpec((er, d_model), lambda i: (i, 0)),
        ],
        out_shape=[
            jax.ShapeDtypeStruct((1, 1, s_dec, s_dec), jnp.float32),
            jax.ShapeDtypeStruct((1, 1, 1, s_enc), jnp.float32),
            jax.ShapeDtypeStruct((n_ehs, d_model), jnp.float32),
        ],
    )(dec_mask, enc_mask, ehs_flat)


@functools.lru_cache(maxsize=None)
def _make_sc_gather(n_tok, d_model, n_posb):
    info = plsc.get_sparse_core_info()
    nc, ns = info.num_cores, info.num_subcores
    nw = nc * ns
    bpw = n_tok // nw
    nq = 4
    gh = bpw // nq
    zpw = n_posb // nw
    L = 16
    mesh = plsc.VectorSubcoreMesh(core_axis_name="c", subcore_axis_name="s")

    @functools.partial(
        pl.kernel,
        mesh=mesh,
        out_type=(
            jax.ShapeDtypeStruct((n_tok, d_model), jnp.float32),
            jax.ShapeDtypeStruct((n_posb,), jnp.float32),
        ),
        scratch_types=[
            pltpu.VMEM((bpw,), jnp.int32),
            pltpu.VMEM((nq, gh, d_model), jnp.float32),
            pltpu.VMEM((zpw,), jnp.float32),
            pltpu.SemaphoreType.DMA((nq,)),
            pltpu.SemaphoreType.DMA((nq,)),
            pltpu.SemaphoreType.DMA,
        ],
    )
    def gather_k(table_hbm, idx_hbm, hid_out, posb_out,
                 idx_v, rows_v, zbuf, sem_g, sem_go, sem_z):
        wid = lax.axis_index("s") * nc + lax.axis_index("c")
        base = wid * bpw
        pltpu.sync_copy(idx_hbm.at[pl.ds(base, bpw)], idx_v)
        gs = [
            pltpu.async_copy(
                table_hbm.at[idx_v.at[pl.ds(q * gh, gh)]], rows_v.at[q],
                sem_g.at[q])
            for q in range(nq)
        ]

        zero = jnp.zeros((L,), dtype=jnp.float32)

        def zfill(c, _):
            zbuf[pl.ds(c * L, L)] = zero
            return 0

        lax.fori_loop(0, zpw // L, zfill, 0)
        zo = pltpu.async_copy(
            zbuf, posb_out.at[pl.ds(wid * zpw, zpw)], sem_z)

        gos = []
        for q in range(nq):
            gs[q].wait()
            gos.append(pltpu.async_copy(
                rows_v.at[q], hid_out.at[pl.ds(base + q * gh, gh)],
                sem_go.at[q]))
        for go in gos:
            go.wait()
        zo.wait()

    return gather_k


def kernel(encoder_hidden_states, encoder_position_bias, decoder_input_ids,
           decoder_attention_mask, encoder_attention_mask, embedding_weight):
    b, s_dec = decoder_input_ids.shape
    vocab, d_model = embedding_weight.shape
    _, s_enc, _ = encoder_hidden_states.shape
    ids_flat = decoder_input_ids.reshape(-1)
    ehs_flat = encoder_hidden_states.reshape(b * s_enc, d_model)
    n_posb = b * NUM_HEADS * s_dec

    gather_k = _make_sc_gather(b * s_dec, d_model, n_posb)
    hid, posb = gather_k(embedding_weight, ids_flat)
    decoder_hidden_states = hid.reshape(b, s_dec, d_model)

    dec_ext, enc_ext, ehs_out = _make_masks(
        decoder_attention_mask, encoder_attention_mask, ehs_flat)
    ehs_out = ehs_out.reshape(encoder_hidden_states.shape)

    decoder_position_bias = posb.reshape(b, NUM_HEADS, s_dec, 1)

    return (ehs_out, encoder_position_bias, decoder_hidden_states,
            enc_ext, dec_ext, decoder_position_bias)


# speedup vs baseline: 1.0013x; 1.0013x over previous
import functools

import jax
import jax.numpy as jnp
from jax import lax
from jax.experimental import pallas as pl
from jax.experimental.pallas import tpu as pltpu
from jax.experimental.pallas import tpu_sc as plsc

NUM_HEADS = 16
NEG = float(jnp.finfo(jnp.float32).min)


def _mask_body(dec_mask_ref, enc_mask_ref, ehs_ref,
               dec_out_ref, enc_out_ref, ehs_out_ref):
    i = pl.program_id(0)
    _, _, R, S = dec_out_ref.shape
    row = i * R + lax.broadcasted_iota(jnp.int32, (1, 1, R, S), 2)
    col = lax.broadcasted_iota(jnp.int32, (1, 1, R, S), 3)
    m = dec_mask_ref[0, :].astype(jnp.float32)[None, None, None, :]
    b = (1.0 - m) * NEG
    dec_out_ref[...] = jnp.where(col <= row, b, NEG)
    e = enc_mask_ref[0, :].astype(jnp.float32)[None, None, None, :]
    enc_out_ref[...] = (1.0 - e) * NEG
    ehs_out_ref[...] = ehs_ref[...]


def _make_masks(dec_mask, enc_mask, ehs_flat):
    _, s_dec = dec_mask.shape
    _, s_enc = enc_mask.shape
    n_ehs, d_model = ehs_flat.shape
    rows_per_step = 512
    grid = s_dec // rows_per_step
    er = n_ehs // grid
    return pl.pallas_call(
        _mask_body,
        grid=(grid,),
        in_specs=[
            pl.BlockSpec((1, s_dec), lambda i: (0, 0)),
            pl.BlockSpec((1, s_enc), lambda i: (0, 0)),
            pl.BlockSpec((er, d_model), lambda i: (i, 0)),
        ],
        out_specs=[
            pl.BlockSpec((1, 1, rows_per_step, s_dec), lambda i: (0, 0, i, 0)),
            pl.BlockSpec((1, 1, 1, s_enc), lambda i: (0, 0, 0, 0)),
            pl.BlockSpec((er, d_model), lambda i: (i, 0)),
        ],
        out_shape=[
            jax.ShapeDtypeStruct((1, 1, s_dec, s_dec), jnp.float32),
            jax.ShapeDtypeStruct((1, 1, 1, s_enc), jnp.float32),
            jax.ShapeDtypeStruct((n_ehs, d_model), jnp.float32),
        ],
    )(dec_mask, enc_mask, ehs_flat)


@functools.lru_cache(maxsize=None)
def _make_sc_gather(n_tok, d_model, n_posb):
    info = plsc.get_sparse_core_info()
    nc, ns = info.num_cores, info.num_subcores
    nw = nc * ns
    bpw = n_tok // nw
    gh = bpw // 2
    zpw = n_posb // nw
    L = 16
    mesh = plsc.VectorSubcoreMesh(core_axis_name="c", subcore_axis_name="s")

    @functools.partial(
        pl.kernel,
        mesh=mesh,
        out_type=(
            jax.ShapeDtypeStruct((n_tok, d_model), jnp.float32),
            jax.ShapeDtypeStruct((n_posb,), jnp.float32),
        ),
        scratch_types=[
            pltpu.VMEM((bpw,), jnp.int32),
            pltpu.VMEM((2, gh, d_model), jnp.float32),
            pltpu.VMEM((zpw,), jnp.float32),
            pltpu.SemaphoreType.DMA((2,)),
            pltpu.SemaphoreType.DMA((2,)),
            pltpu.SemaphoreType.DMA,
        ],
    )
    def gather_k(table_hbm, idx_hbm, hid_out, posb_out,
                 idx_v, rows_v, zbuf, sem_g, sem_go, sem_z):
        wid = lax.axis_index("s") * nc + lax.axis_index("c")
        base = wid * bpw
        pltpu.sync_copy(idx_hbm.at[pl.ds(base, bpw)], idx_v)
        g0 = pltpu.async_copy(
            table_hbm.at[idx_v.at[pl.ds(0, gh)]], rows_v.at[0], sem_g.at[0])
        g1 = pltpu.async_copy(
            table_hbm.at[idx_v.at[pl.ds(gh, gh)]], rows_v.at[1], sem_g.at[1])

        zero = jnp.zeros((L,), dtype=jnp.float32)

        def zfill(c, _):
            zbuf[pl.ds(c * L, L)] = zero
            return 0

        lax.fori_loop(0, zpw // L, zfill, 0)
        zo = pltpu.async_copy(
            zbuf, posb_out.at[pl.ds(wid * zpw, zpw)], sem_z)

        g0.wait()
        go0 = pltpu.async_copy(
            rows_v.at[0], hid_out.at[pl.ds(base, gh)], sem_go.at[0])
        g1.wait()
        go1 = pltpu.async_copy(
            rows_v.at[1], hid_out.at[pl.ds(base + gh, gh)], sem_go.at[1])
        go0.wait()
        go1.wait()
        zo.wait()

    return gather_k


def kernel(encoder_hidden_states, encoder_position_bias, decoder_input_ids,
           decoder_attention_mask, encoder_attention_mask, embedding_weight):
    b, s_dec = decoder_input_ids.shape
    vocab, d_model = embedding_weight.shape
    _, s_enc, _ = encoder_hidden_states.shape
    ids_flat = decoder_input_ids.reshape(-1)
    ehs_flat = encoder_hidden_states.reshape(b * s_enc, d_model)
    n_posb = b * NUM_HEADS * s_dec

    gather_k = _make_sc_gather(b * s_dec, d_model, n_posb)
    hid, posb = gather_k(embedding_weight, ids_flat)
    decoder_hidden_states = hid.reshape(b, s_dec, d_model)

    dec_ext, enc_ext, ehs_out = _make_masks(
        decoder_attention_mask, encoder_attention_mask, ehs_flat)
    ehs_out = ehs_out.reshape(encoder_hidden_states.shape)

    decoder_position_bias = posb.reshape(b, NUM_HEADS, s_dec, 1)

    return (ehs_out, encoder_position_bias, decoder_hidden_states,
            enc_ext, dec_ext, decoder_position_bias)
